# baseline (speedup 1.0000x reference)
import functools

import jax
import jax.numpy as jnp
from jax import lax
from jax.experimental import pallas as pl
from jax.experimental.pallas import tpu as pltpu
from jax.experimental.pallas import tpu_sc as plsc

NN = 10000
EE = 320000
DIN = 128
HID = 64
NOUT = 16
NR = 4
NG = 64

NC = 2
NS = 16
NW = NC * NS
CHUNK = 128
CPW = 80
CPW0 = 114
CPW1 = 46
TOTCH = NS * (CPW0 + CPW1)
EPW = CPW * CHUNK
EPAD = NW * EPW
NACC = 10240
RPT = NACC // NS

BN = 400
NB = NN // BN

@functools.lru_cache(maxsize=None)
def _mesh():
    return plsc.VectorSubcoreMesh(core_axis_name="c", subcore_axis_name="s")


ZR = 64


def _zero_acc(zbuf, acc_sh, sid):
    def zrow(j, _):
        for c in range(HID // 16):
            zbuf[j, pl.ds(c * 16, 16)] = jnp.zeros((16,), jnp.float32)
        return 0
    lax.fori_loop(0, ZR, zrow, 0)
    for q in range(RPT // ZR):
        pltpu.sync_copy(zbuf, acc_sh.at[pl.ds(sid * RPT + q * ZR, ZR)])


NCR = NACC * NR
RPTC = NCR // NS


def _cnt_body(fgc_hbm, out_hbm, fg_v, ones, zbuf, acc_sh):
    cid = lax.axis_index("c")
    sid = lax.axis_index("s")
    wid = sid * NC + cid

    def zrow(j, _):
        zbuf[j, :] = jnp.zeros((16,), jnp.float32)
        return 0
    lax.fori_loop(0, ZR, zrow, 0)

    def orow(j, _):
        ones[j, :] = jnp.full((16,), 1.0, jnp.float32)
        return 0
    lax.fori_loop(0, CHUNK, orow, 0)

    base = sid * RPTC
    for q in range(RPTC // ZR):
        pltpu.sync_copy(zbuf, acc_sh.at[pl.ds(base + q * ZR, ZR)])
    pltpu.sync_copy(fgc_hbm.at[wid], fg_v)
    plsc.subcore_barrier()

    def chunk(k, _):
        pltpu.sync_copy(ones, acc_sh.at[fg_v.at[k]], add=True)
        return 0
    lax.fori_loop(0, CPW, chunk, 0)

    plsc.subcore_barrier()
    pltpu.sync_copy(acc_sh.at[pl.ds(sid * RPTC, RPTC)],
                    out_hbm.at[cid, pl.ds(sid * RPTC, RPTC)])


def _cnt_call():
    return pl.kernel(
        _cnt_body, mesh=_mesh(),
    out_type=jax.ShapeDtypeStruct((NC, NCR, 16), jnp.float32),
    compiler_params=pltpu.CompilerParams(use_tc_tiling_on_sc=False),
    scratch_types=[
        pltpu.VMEM((CPW, CHUNK), jnp.int32),
        pltpu.VMEM((CHUNK, 16), jnp.float32),
        pltpu.VMEM((ZR, 16), jnp.float32),
        pltpu.VMEM_SHARED((NCR, 16), jnp.float32),
    ],
)


def _edge_body(lin_hbm, f_hbm, xg_hbm, fg_hbm, d_hbm, out_hbm,
               xg_v, fg_v, d_v, xrows, frows, msg, zbuf, acc_sh,
               sem1, sem2, sem3, sem4):
    cid = lax.axis_index("c")
    sid = lax.axis_index("s")

    _zero_acc(zbuf, acc_sh, sid)

    H = CHUNK // 2

    def run(base_ch, cpwc):
        pltpu.sync_copy(xg_hbm.at[pl.ds(base_ch, cpwc)],
                        xg_v.at[pl.ds(0, cpwc)])
        pltpu.sync_copy(fg_hbm.at[pl.ds(base_ch, cpwc)],
                        fg_v.at[pl.ds(0, cpwc)])
        pltpu.sync_copy(d_hbm.at[pl.ds(base_ch, cpwc)],
                        d_v.at[pl.ds(0, cpwc)])
        plsc.subcore_barrier()

        def chunk(k, _):
            c1 = pltpu.async_copy(lin_hbm.at[xg_v.at[k, pl.ds(0, H)]],
                                  xrows.at[pl.ds(0, H)], sem1)
            c2 = pltpu.async_copy(lin_hbm.at[xg_v.at[k, pl.ds(H, H)]],
                                  xrows.at[pl.ds(H, H)], sem2)
            c3 = pltpu.async_copy(f_hbm.at[fg_v.at[k, pl.ds(0, H)]],
                                  frows.at[pl.ds(0, H)], sem3)
            c4 = pltpu.async_copy(f_hbm.at[fg_v.at[k, pl.ds(H, H)]],
                                  frows.at[pl.ds(H, H)], sem4)
            def ebody(j, _):
                for c in range(4):
                    xv = xrows[j, pl.ds(c * 16, 16)]
                    bv = frows[j, pl.ds(c * 16, 16)]
                    gv = frows[j, pl.ds(64 + c * 16, 16)]
                    msg[j, pl.ds(c * 16, 16)] = jnp.maximum(gv * xv + bv, 0.0)
                return 0
            c1.wait()
            c3.wait()
            lax.fori_loop(0, H, ebody, 0)
            c2.wait()
            c4.wait()
            lax.fori_loop(H, CHUNK, ebody, 0)
            pltpu.sync_copy(msg, acc_sh.at[d_v.at[k]], add=True)
            return 0
        lax.fori_loop(0, cpwc, chunk, 0)

    @pl.when(cid == 0)
    def _():
        run(sid * CPW0, CPW0)

    @pl.when(cid == 1)
    def _():
        run(NS * CPW0 + sid * CPW1, CPW1)

    plsc.subcore_barrier()
    pltpu.sync_copy(acc_sh.at[pl.ds(sid * RPT, RPT)],
                    out_hbm.at[cid, pl.ds(sid * RPT, RPT)])


def _edge_call():
    return pl.kernel(
        _edge_body, mesh=_mesh(),
    out_type=jax.ShapeDtypeStruct((NC, NACC, HID), jnp.float32),
    compiler_params=pltpu.CompilerParams(use_tc_tiling_on_sc=False),
    scratch_types=[
        pltpu.VMEM((CPW0, CHUNK), jnp.int32),
        pltpu.VMEM((CPW0, CHUNK), jnp.int32),
        pltpu.VMEM((CPW0, CHUNK), jnp.int32),
        pltpu.VMEM((CHUNK, HID), jnp.float32),
        pltpu.VMEM((CHUNK, 2 * HID), jnp.float32),
        pltpu.VMEM((CHUNK, HID), jnp.float32),
        pltpu.VMEM((ZR, HID), jnp.float32),
        pltpu.VMEM_SHARED((NACC, HID), jnp.float32),
        pltpu.SemaphoreType.DMA,
        pltpu.SemaphoreType.DMA,
        pltpu.SemaphoreType.DMA,
        pltpu.SemaphoreType.DMA,
    ],
)


def _tables_common(h, cnt_ref, wall_ref, bf_ref, lin_ref, f_ref, skip_ref):
    t = jnp.dot(h, wall_ref[...], preferred_element_type=jnp.float32)
    lin_ref[...] = t[:, :NR * HID]
    fs = t[:, 768:896]
    sk = t[:, 896:960]
    skip_ref[...] = jnp.maximum(fs[:, HID:] * sk + fs[:, :HID], 0.0)
    cnt = cnt_ref[0] + cnt_ref[1]
    for r in range(NR):
        invr = 1.0 / jnp.maximum(cnt[:, r:r + 1], 1.0)
        lo = NR * HID + r * 2 * HID
        f_ref[:, r * 2 * HID:(r + 1) * 2 * HID] = (
            t[:, lo:lo + 2 * HID] + bf_ref[0, r * 2 * HID:(r + 1) * 2 * HID]
        ) * invr


def _layer0_body(x_ref, cnt_ref, wenc_ref, benc_ref, wall_ref, bf_ref,
                 lin_ref, f_ref, skip_ref):
    h = jnp.dot(x_ref[...], wenc_ref[...],
                preferred_element_type=jnp.float32) + benc_ref[...]
    _tables_common(h, cnt_ref, wall_ref, bf_ref, lin_ref, f_ref, skip_ref)


def _layer1_body(skin_ref, acc_ref, cnt_ref, bns_ref, bnb_ref, wall_ref,
                 bf_ref, lin_ref, f_ref, skip_ref):
    h = skin_ref[...] + acc_ref[0] + acc_ref[1]
    h = jnp.maximum(h * bns_ref[...] + bnb_ref[...], 0.0)
    _tables_common(h, cnt_ref, wall_ref, bf_ref, lin_ref, f_ref, skip_ref)


def _head_body(skin_ref, acc_ref, batch_ref, wl_ref, bl_ref, wc_ref, bc_ref,
               out_ref, accm_ref):
    i = pl.program_id(0)

    @pl.when(i == 0)
    def _():
        accm_ref[...] = jnp.zeros((NG, HID), jnp.float32)

    y = skin_ref[...] + acc_ref[0] + acc_ref[1]
    b = batch_ref[0]
    oh = (b == lax.broadcasted_iota(jnp.int32, (NG, BN), 0)).astype(jnp.float32)
    accm_ref[...] += jnp.dot(oh, y, preferred_element_type=jnp.float32)

    @pl.when(i == NB - 1)
    def _():
        g = jnp.maximum(
            jnp.dot(accm_ref[...], wl_ref[...],
                    preferred_element_type=jnp.float32) + bl_ref[...], 0.0)
        out_ref[...] = jnp.dot(g, wc_ref[...],
                               preferred_element_type=jnp.float32) + bc_ref[...]


def _fixed(shape):
    return pl.BlockSpec(shape, lambda i: tuple(0 for _ in shape))


_tables_out = [
    jax.ShapeDtypeStruct((NN, NR * HID), jnp.float32),
    jax.ShapeDtypeStruct((NN, NR * 2 * HID), jnp.float32),
    jax.ShapeDtypeStruct((NN, HID), jnp.float32),
]
_tables_out_specs = [
    pl.BlockSpec((BN, NR * HID), lambda i: (i, 0)),
    pl.BlockSpec((BN, NR * 2 * HID), lambda i: (i, 0)),
    pl.BlockSpec((BN, HID), lambda i: (i, 0)),
]

_layer0_call = pl.pallas_call(
    _layer0_body,
    grid=(NB,),
    in_specs=[
        pl.BlockSpec((BN, DIN), lambda i: (i, 0)),
        pl.BlockSpec((NC, BN, 4), lambda i: (0, i, 0)),
        _fixed((DIN, HID)),
        _fixed((1, HID)),
        _fixed((HID, 960)),
        _fixed((1, NR * 2 * HID)),
    ],
    out_specs=_tables_out_specs,
    out_shape=_tables_out,
)

_layer1_call = pl.pallas_call(
    _layer1_body,
    grid=(NB,),
    in_specs=[
        pl.BlockSpec((BN, HID), lambda i: (i, 0)),
        pl.BlockSpec((NC, BN, HID), lambda i: (0, i, 0)),
        pl.BlockSpec((NC, BN, 4), lambda i: (0, i, 0)),
        _fixed((1, HID)),
        _fixed((1, HID)),
        _fixed((HID, 960)),
        _fixed((1, NR * 2 * HID)),
    ],
    out_specs=_tables_out_specs,
    out_shape=_tables_out,
)

_head_call = pl.pallas_call(
    _head_body,
    grid=(NB,),
    in_specs=[
        pl.BlockSpec((BN, HID), lambda i: (i, 0)),
        pl.BlockSpec((NC, BN, HID), lambda i: (0, i, 0)),
        pl.BlockSpec((1, 1, BN), lambda i: (i, 0, 0)),
        _fixed((HID, HID)),
        _fixed((1, HID)),
        _fixed((HID, NOUT)),
        _fixed((1, NOUT)),
    ],
    out_specs=pl.BlockSpec((NG, NOUT), lambda i: (0, 0)),
    out_shape=jax.ShapeDtypeStruct((NG, NOUT), jnp.float32),
    scratch_shapes=[pltpu.VMEM((NG, HID), jnp.float32)],
)


def _wall(p, l):
    wcat = jnp.concatenate([p['W_rel_%d' % l][r] for r in range(NR)], axis=1)
    wfcat = jnp.concatenate([p['Wf_rel_%d' % l][r] for r in range(NR)], axis=1)
    wall = jnp.concatenate(
        [wcat, wfcat, p['Wf_skip_%d' % l], p['W_skip_%d' % l]], axis=1)
    bfcat = jnp.concatenate([p['bf_rel_%d' % l][r] for r in range(NR)], axis=0)
    return wall, bfcat.reshape(1, NR * 2 * HID)


def kernel(x, edge_index, edge_type, batch, params):
    p = params
    src = edge_index[0].astype(jnp.int32)
    dst = edge_index[1].astype(jnp.int32)
    et = edge_type.astype(jnp.int32)

    pad = EPAD - EE
    zpad = jnp.zeros((pad,), jnp.int32)
    xg = jnp.concatenate([src * NR + et, zpad]).reshape(TOTCH, CHUNK)
    fg = jnp.concatenate([dst * NR + et, zpad]).reshape(TOTCH, CHUNK)
    d3 = jnp.concatenate([dst, jnp.full((pad,), NN, jnp.int32)]
                         ).reshape(TOTCH, CHUNK)
    dpad = jnp.concatenate([dst, jnp.full((pad,), NN, jnp.int32)])
    etpad = jnp.concatenate([et, zpad])
    fgc3 = (dpad * NR + etpad).reshape(NW, CPW, CHUNK)

    cntf = _cnt_call()(fgc3)
    cnt2 = cntf[:, :NN * NR, 0].reshape(NC, NN, NR)

    wall0, bf0 = _wall(p, 0)
    wall1, bf1 = _wall(p, 1)
    bns = (p['bn_w'] / jnp.sqrt(p['bn_rv'] + 1e-5)).reshape(1, HID)
    bnb = (p['bn_b'] - p['bn_rm'] * bns[0]).reshape(1, HID)

    lin_t, f_t, skip0 = _layer0_call(
        x, cnt2, p['W_enc'], p['b_enc'].reshape(1, HID), wall0, bf0)
    acc0 = _edge_call()(lin_t.reshape(NN * NR, HID),
                        f_t.reshape(NN * NR, 2 * HID), xg, fg, d3)

    lin_t1, f_t1, skip1 = _layer1_call(
        skip0, acc0[:, :NN, :HID], cnt2, bns, bnb, wall1, bf1)
    acc1 = _edge_call()(lin_t1.reshape(NN * NR, HID),
                        f_t1.reshape(NN * NR, 2 * HID), xg, fg, d3)

    out = _head_call(skip1, acc1[:, :NN, :HID],
                     batch.astype(jnp.int32).reshape(NB, 1, BN),
                     p['W_lin'], p['b_lin'].reshape(1, HID),
                     p['W_clf'], p['b_clf'].reshape(1, NOUT))
    return out


# speedup vs baseline: 1.0285x; 1.0285x over previous
import functools

import jax
import jax.numpy as jnp
from jax import lax
from jax.experimental import pallas as pl
from jax.experimental.pallas import tpu as pltpu
from jax.experimental.pallas import tpu_sc as plsc

NN = 10000
EE = 320000
DIN = 128
HID = 64
NOUT = 16
NR = 4
NG = 64

NC = 2
NS = 16
NW = NC * NS
CHUNK = 128
CPW = 80
CPW0 = 110
CPW1 = 50
TOTCH = NS * (CPW0 + CPW1)
EPW = CPW * CHUNK
EPAD = NW * EPW
NACC = 10240
RPT = NACC // NS

BN = 400
NB = NN // BN

@functools.lru_cache(maxsize=None)
def _mesh():
    return plsc.VectorSubcoreMesh(core_axis_name="c", subcore_axis_name="s")


ZR = 64


def _zero_acc(zbuf, acc_sh, sid):
    def zrow(j, _):
        for c in range(HID // 16):
            zbuf[j, pl.ds(c * 16, 16)] = jnp.zeros((16,), jnp.float32)
        return 0
    lax.fori_loop(0, ZR, zrow, 0)
    for q in range(RPT // ZR):
        pltpu.sync_copy(zbuf, acc_sh.at[pl.ds(sid * RPT + q * ZR, ZR)])


NCR = NACC * NR
RPTC = NCR // NS


def _cnt_body(fgc_hbm, out_hbm, fg_v, ones, zbuf, acc_sh):
    cid = lax.axis_index("c")
    sid = lax.axis_index("s")
    wid = sid * NC + cid

    def zrow(j, _):
        zbuf[j, :] = jnp.zeros((16,), jnp.float32)
        return 0
    lax.fori_loop(0, ZR, zrow, 0)

    def orow(j, _):
        ones[j, :] = jnp.full((16,), 1.0, jnp.float32)
        return 0
    lax.fori_loop(0, CHUNK, orow, 0)

    base = sid * RPTC
    for q in range(RPTC // ZR):
        pltpu.sync_copy(zbuf, acc_sh.at[pl.ds(base + q * ZR, ZR)])
    pltpu.sync_copy(fgc_hbm.at[wid], fg_v)
    plsc.subcore_barrier()

    def chunk(k, _):
        pltpu.sync_copy(ones, acc_sh.at[fg_v.at[k]], add=True)
        return 0
    lax.fori_loop(0, CPW, chunk, 0)

    plsc.subcore_barrier()
    pltpu.sync_copy(acc_sh.at[pl.ds(sid * RPTC, RPTC)],
                    out_hbm.at[cid, pl.ds(sid * RPTC, RPTC)])


def _cnt_call():
    return pl.kernel(
        _cnt_body, mesh=_mesh(),
    out_type=jax.ShapeDtypeStruct((NC, NCR, 16), jnp.float32),
    compiler_params=pltpu.CompilerParams(use_tc_tiling_on_sc=False),
    scratch_types=[
        pltpu.VMEM((CPW, CHUNK), jnp.int32),
        pltpu.VMEM((CHUNK, 16), jnp.float32),
        pltpu.VMEM((ZR, 16), jnp.float32),
        pltpu.VMEM_SHARED((NCR, 16), jnp.float32),
    ],
)


def _edge_body(lin_hbm, f_hbm, xg_hbm, fg_hbm, d_hbm, out_hbm,
               xg_v, fg_v, d_v, xrows, frows, msg, zbuf, acc_sh,
               sem1, sem2, sem3, sem4):
    cid = lax.axis_index("c")
    sid = lax.axis_index("s")

    _zero_acc(zbuf, acc_sh, sid)

    H = CHUNK // 2

    def run(base_ch, cpwc):
        pltpu.sync_copy(xg_hbm.at[pl.ds(base_ch, cpwc)],
                        xg_v.at[pl.ds(0, cpwc)])
        pltpu.sync_copy(fg_hbm.at[pl.ds(base_ch, cpwc)],
                        fg_v.at[pl.ds(0, cpwc)])
        pltpu.sync_copy(d_hbm.at[pl.ds(base_ch, cpwc)],
                        d_v.at[pl.ds(0, cpwc)])
        plsc.subcore_barrier()

        def chunk(k, _):
            c1 = pltpu.async_copy(lin_hbm.at[xg_v.at[k, pl.ds(0, H)]],
                                  xrows.at[pl.ds(0, H)], sem1)
            c2 = pltpu.async_copy(lin_hbm.at[xg_v.at[k, pl.ds(H, H)]],
                                  xrows.at[pl.ds(H, H)], sem2)
            c3 = pltpu.async_copy(f_hbm.at[fg_v.at[k, pl.ds(0, H)]],
                                  frows.at[pl.ds(0, H)], sem3)
            c4 = pltpu.async_copy(f_hbm.at[fg_v.at[k, pl.ds(H, H)]],
                                  frows.at[pl.ds(H, H)], sem4)
            def ebody(j, _):
                for c in range(4):
                    xv = xrows[j, pl.ds(c * 16, 16)]
                    bv = frows[j, pl.ds(c * 16, 16)]
                    gv = frows[j, pl.ds(64 + c * 16, 16)]
                    msg[j, pl.ds(c * 16, 16)] = jnp.maximum(gv * xv + bv, 0.0)
                return 0
            c1.wait()
            c3.wait()
            lax.fori_loop(0, H, ebody, 0)
            c2.wait()
            c4.wait()
            lax.fori_loop(H, CHUNK, ebody, 0)
            pltpu.sync_copy(msg, acc_sh.at[d_v.at[k]], add=True)
            return 0
        lax.fori_loop(0, cpwc, chunk, 0)

    @pl.when(cid == 0)
    def _():
        run(sid * CPW0, CPW0)

    @pl.when(cid == 1)
    def _():
        run(NS * CPW0 + sid * CPW1, CPW1)

    plsc.subcore_barrier()
    pltpu.sync_copy(acc_sh.at[pl.ds(sid * RPT, RPT)],
                    out_hbm.at[cid, pl.ds(sid * RPT, RPT)])


def _edge_call():
    return pl.kernel(
        _edge_body, mesh=_mesh(),
    out_type=jax.ShapeDtypeStruct((NC, NACC, HID), jnp.float32),
    compiler_params=pltpu.CompilerParams(use_tc_tiling_on_sc=False),
    scratch_types=[
        pltpu.VMEM((CPW0, CHUNK), jnp.int32),
        pltpu.VMEM((CPW0, CHUNK), jnp.int32),
        pltpu.VMEM((CPW0, CHUNK), jnp.int32),
        pltpu.VMEM((CHUNK, HID), jnp.float32),
        pltpu.VMEM((CHUNK, 2 * HID), jnp.float32),
        pltpu.VMEM((CHUNK, HID), jnp.float32),
        pltpu.VMEM((ZR, HID), jnp.float32),
        pltpu.VMEM_SHARED((NACC, HID), jnp.float32),
        pltpu.SemaphoreType.DMA,
        pltpu.SemaphoreType.DMA,
        pltpu.SemaphoreType.DMA,
        pltpu.SemaphoreType.DMA,
    ],
)


def _tables_common(h, cnt_ref, wall_ref, bf_ref, lin_ref, f_ref, skip_ref):
    t = jnp.dot(h, wall_ref[...], preferred_element_type=jnp.float32)
    lin_ref[...] = t[:, :NR * HID]
    fs = t[:, 768:896]
    sk = t[:, 896:960]
    skip_ref[...] = jnp.maximum(fs[:, HID:] * sk + fs[:, :HID], 0.0)
    cnt = cnt_ref[0] + cnt_ref[1]
    for r in range(NR):
        invr = 1.0 / jnp.maximum(cnt[:, r:r + 1], 1.0)
        lo = NR * HID + r * 2 * HID
        f_ref[:, r * 2 * HID:(r + 1) * 2 * HID] = (
            t[:, lo:lo + 2 * HID] + bf_ref[0, r * 2 * HID:(r + 1) * 2 * HID]
        ) * invr


def _layer0_body(x_ref, cnt_ref, wenc_ref, benc_ref, wall_ref, bf_ref,
                 lin_ref, f_ref, skip_ref):
    h = jnp.dot(x_ref[...], wenc_ref[...],
                preferred_element_type=jnp.float32) + benc_ref[...]
    _tables_common(h, cnt_ref, wall_ref, bf_ref, lin_ref, f_ref, skip_ref)


def _layer1_body(skin_ref, acc_ref, cnt_ref, bns_ref, bnb_ref, wall_ref,
                 bf_ref, lin_ref, f_ref, skip_ref):
    h = skin_ref[...] + acc_ref[0] + acc_ref[1]
    h = jnp.maximum(h * bns_ref[...] + bnb_ref[...], 0.0)
    _tables_common(h, cnt_ref, wall_ref, bf_ref, lin_ref, f_ref, skip_ref)


def _head_body(skin_ref, acc_ref, batch_ref, wl_ref, bl_ref, wc_ref, bc_ref,
               out_ref, accm_ref):
    i = pl.program_id(0)

    @pl.when(i == 0)
    def _():
        accm_ref[...] = jnp.zeros((NG, HID), jnp.float32)

    y = skin_ref[...] + acc_ref[0] + acc_ref[1]
    b = batch_ref[0]
    oh = (b == lax.broadcasted_iota(jnp.int32, (NG, BN), 0)).astype(jnp.float32)
    accm_ref[...] += jnp.dot(oh, y, preferred_element_type=jnp.float32)

    @pl.when(i == NB - 1)
    def _():
        g = jnp.maximum(
            jnp.dot(accm_ref[...], wl_ref[...],
                    preferred_element_type=jnp.float32) + bl_ref[...], 0.0)
        out_ref[...] = jnp.dot(g, wc_ref[...],
                               preferred_element_type=jnp.float32) + bc_ref[...]


def _fixed(shape):
    return pl.BlockSpec(shape, lambda i: tuple(0 for _ in shape))


_tables_out = [
    jax.ShapeDtypeStruct((NN, NR * HID), jnp.float32),
    jax.ShapeDtypeStruct((NN, NR * 2 * HID), jnp.float32),
    jax.ShapeDtypeStruct((NN, HID), jnp.float32),
]
_tables_out_specs = [
    pl.BlockSpec((BN, NR * HID), lambda i: (i, 0)),
    pl.BlockSpec((BN, NR * 2 * HID), lambda i: (i, 0)),
    pl.BlockSpec((BN, HID), lambda i: (i, 0)),
]

_layer0_call = pl.pallas_call(
    _layer0_body,
    grid=(NB,),
    in_specs=[
        pl.BlockSpec((BN, DIN), lambda i: (i, 0)),
        pl.BlockSpec((NC, BN, 4), lambda i: (0, i, 0)),
        _fixed((DIN, HID)),
        _fixed((1, HID)),
        _fixed((HID, 960)),
        _fixed((1, NR * 2 * HID)),
    ],
    out_specs=_tables_out_specs,
    out_shape=_tables_out,
)

_layer1_call = pl.pallas_call(
    _layer1_body,
    grid=(NB,),
    in_specs=[
        pl.BlockSpec((BN, HID), lambda i: (i, 0)),
        pl.BlockSpec((NC, BN, HID), lambda i: (0, i, 0)),
        pl.BlockSpec((NC, BN, 4), lambda i: (0, i, 0)),
        _fixed((1, HID)),
        _fixed((1, HID)),
        _fixed((HID, 960)),
        _fixed((1, NR * 2 * HID)),
    ],
    out_specs=_tables_out_specs,
    out_shape=_tables_out,
)

_head_call = pl.pallas_call(
    _head_body,
    grid=(NB,),
    in_specs=[
        pl.BlockSpec((BN, HID), lambda i: (i, 0)),
        pl.BlockSpec((NC, BN, HID), lambda i: (0, i, 0)),
        pl.BlockSpec((1, 1, BN), lambda i: (i, 0, 0)),
        _fixed((HID, HID)),
        _fixed((1, HID)),
        _fixed((HID, NOUT)),
        _fixed((1, NOUT)),
    ],
    out_specs=pl.BlockSpec((NG, NOUT), lambda i: (0, 0)),
    out_shape=jax.ShapeDtypeStruct((NG, NOUT), jnp.float32),
    scratch_shapes=[pltpu.VMEM((NG, HID), jnp.float32)],
)


def _wall(p, l):
    wcat = jnp.concatenate([p['W_rel_%d' % l][r] for r in range(NR)], axis=1)
    wfcat = jnp.concatenate([p['Wf_rel_%d' % l][r] for r in range(NR)], axis=1)
    wall = jnp.concatenate(
        [wcat, wfcat, p['Wf_skip_%d' % l], p['W_skip_%d' % l]], axis=1)
    bfcat = jnp.concatenate([p['bf_rel_%d' % l][r] for r in range(NR)], axis=0)
    return wall, bfcat.reshape(1, NR * 2 * HID)


def kernel(x, edge_index, edge_type, batch, params):
    p = params
    src = edge_index[0].astype(jnp.int32)
    dst = edge_index[1].astype(jnp.int32)
    et = edge_type.astype(jnp.int32)

    pad = EPAD - EE
    zpad = jnp.zeros((pad,), jnp.int32)
    xg = jnp.concatenate([src * NR + et, zpad]).reshape(TOTCH, CHUNK)
    fg = jnp.concatenate([dst * NR + et, zpad]).reshape(TOTCH, CHUNK)
    d3 = jnp.concatenate([dst, jnp.full((pad,), NN, jnp.int32)]
                         ).reshape(TOTCH, CHUNK)
    dpad = jnp.concatenate([dst, jnp.full((pad,), NN, jnp.int32)])
    etpad = jnp.concatenate([et, zpad])
    fgc3 = (dpad * NR + etpad).reshape(NW, CPW, CHUNK)

    cntf = _cnt_call()(fgc3)
    cnt2 = cntf[:, :NN * NR, 0].reshape(NC, NN, NR)

    wall0, bf0 = _wall(p, 0)
    wall1, bf1 = _wall(p, 1)
    bns = (p['bn_w'] / jnp.sqrt(p['bn_rv'] + 1e-5)).reshape(1, HID)
    bnb = (p['bn_b'] - p['bn_rm'] * bns[0]).reshape(1, HID)

    lin_t, f_t, skip0 = _layer0_call(
        x, cnt2, p['W_enc'], p['b_enc'].reshape(1, HID), wall0, bf0)
    acc0 = _edge_call()(lin_t.reshape(NN * NR, HID),
                        f_t.reshape(NN * NR, 2 * HID), xg, fg, d3)

    lin_t1, f_t1, skip1 = _layer1_call(
        skip0, acc0[:, :NN, :HID], cnt2, bns, bnb, wall1, bf1)
    acc1 = _edge_call()(lin_t1.reshape(NN * NR, HID),
                        f_t1.reshape(NN * NR, 2 * HID), xg, fg, d3)

    out = _head_call(skip1, acc1[:, :NN, :HID],
                     batch.astype(jnp.int32).reshape(NB, 1, BN),
                     p['W_lin'], p['b_lin'].reshape(1, HID),
                     p['W_clf'], p['b_clf'].reshape(1, NOUT))
    return out


# speedup vs baseline: 1.0437x; 1.0148x over previous
import functools

import jax
import jax.numpy as jnp
from jax import lax
from jax.experimental import pallas as pl
from jax.experimental.pallas import tpu as pltpu
from jax.experimental.pallas import tpu_sc as plsc

NN = 10000
EE = 320000
DIN = 128
HID = 64
NOUT = 16
NR = 4
NG = 64

NC = 2
NS = 16
NW = NC * NS
CHUNK = 128
CPW = 80
CPW0 = 108
CPW1 = 52
TOTCH = NS * (CPW0 + CPW1)
EPW = CPW * CHUNK
EPAD = NW * EPW
NACC = 10240
RPT = NACC // NS

BN = 400
NB = NN // BN

@functools.lru_cache(maxsize=None)
def _mesh():
    return plsc.VectorSubcoreMesh(core_axis_name="c", subcore_axis_name="s")


ZR = 64


def _zero_acc(zbuf, acc_sh, sid):
    def zrow(j, _):
        for c in range(HID // 16):
            zbuf[j, pl.ds(c * 16, 16)] = jnp.zeros((16,), jnp.float32)
        return 0
    lax.fori_loop(0, ZR, zrow, 0)
    for q in range(RPT // ZR):
        pltpu.sync_copy(zbuf, acc_sh.at[pl.ds(sid * RPT + q * ZR, ZR)])


NCR = NACC * NR
RPTC = NCR // NS


def _cnt_body(fgc_hbm, out_hbm, fg_v, ones, zbuf, acc_sh):
    cid = lax.axis_index("c")
    sid = lax.axis_index("s")
    wid = sid * NC + cid

    def zrow(j, _):
        zbuf[j, :] = jnp.zeros((16,), jnp.float32)
        return 0
    lax.fori_loop(0, ZR, zrow, 0)

    def orow(j, _):
        ones[j, :] = jnp.full((16,), 1.0, jnp.float32)
        return 0
    lax.fori_loop(0, CHUNK, orow, 0)

    base = sid * RPTC
    for q in range(RPTC // ZR):
        pltpu.sync_copy(zbuf, acc_sh.at[pl.ds(base + q * ZR, ZR)])
    pltpu.sync_copy(fgc_hbm.at[wid], fg_v)
    plsc.subcore_barrier()

    def chunk(k, _):
        pltpu.sync_copy(ones, acc_sh.at[fg_v.at[k]], add=True)
        return 0
    lax.fori_loop(0, CPW, chunk, 0)

    plsc.subcore_barrier()
    pltpu.sync_copy(acc_sh.at[pl.ds(sid * RPTC, RPTC)],
                    out_hbm.at[cid, pl.ds(sid * RPTC, RPTC)])


def _cnt_call():
    return pl.kernel(
        _cnt_body, mesh=_mesh(),
    out_type=jax.ShapeDtypeStruct((NC, NCR, 16), jnp.float32),
    compiler_params=pltpu.CompilerParams(use_tc_tiling_on_sc=False),
    scratch_types=[
        pltpu.VMEM((CPW, CHUNK), jnp.int32),
        pltpu.VMEM((CHUNK, 16), jnp.float32),
        pltpu.VMEM((ZR, 16), jnp.float32),
        pltpu.VMEM_SHARED((NCR, 16), jnp.float32),
    ],
)


def _edge_body(lin_hbm, f_hbm, xg_hbm, fg_hbm, d_hbm, out_hbm,
               xg_v, fg_v, d_v, xrows, frows, msg, zbuf, acc_sh,
               sem1, sem2, sem3, sem4):
    cid = lax.axis_index("c")
    sid = lax.axis_index("s")

    _zero_acc(zbuf, acc_sh, sid)

    H = CHUNK // 2

    def run(base_ch, cpwc):
        pltpu.sync_copy(xg_hbm.at[pl.ds(base_ch, cpwc)],
                        xg_v.at[pl.ds(0, cpwc)])
        pltpu.sync_copy(fg_hbm.at[pl.ds(base_ch, cpwc)],
                        fg_v.at[pl.ds(0, cpwc)])
        pltpu.sync_copy(d_hbm.at[pl.ds(base_ch, cpwc)],
                        d_v.at[pl.ds(0, cpwc)])
        plsc.subcore_barrier()

        def chunk(k, _):
            c1 = pltpu.async_copy(lin_hbm.at[xg_v.at[k, pl.ds(0, H)]],
                                  xrows.at[pl.ds(0, H)], sem1)
            c2 = pltpu.async_copy(lin_hbm.at[xg_v.at[k, pl.ds(H, H)]],
                                  xrows.at[pl.ds(H, H)], sem2)
            c3 = pltpu.async_copy(f_hbm.at[fg_v.at[k, pl.ds(0, H)]],
                                  frows.at[pl.ds(0, H)], sem3)
            c4 = pltpu.async_copy(f_hbm.at[fg_v.at[k, pl.ds(H, H)]],
                                  frows.at[pl.ds(H, H)], sem4)
            def ebody(j, _):
                for c in range(4):
                    xv = xrows[j, pl.ds(c * 16, 16)]
                    bv = frows[j, pl.ds(c * 16, 16)]
                    gv = frows[j, pl.ds(64 + c * 16, 16)]
                    msg[j, pl.ds(c * 16, 16)] = jnp.maximum(gv * xv + bv, 0.0)
                return 0
            c1.wait()
            c3.wait()
            lax.fori_loop(0, H, ebody, 0)
            c2.wait()
            c4.wait()
            lax.fori_loop(H, CHUNK, ebody, 0)
            pltpu.sync_copy(msg, acc_sh.at[d_v.at[k]], add=True)
            return 0
        lax.fori_loop(0, cpwc, chunk, 0)

    @pl.when(cid == 0)
    def _():
        run(sid * CPW0, CPW0)

    @pl.when(cid == 1)
    def _():
        run(NS * CPW0 + sid * CPW1, CPW1)

    plsc.subcore_barrier()
    pltpu.sync_copy(acc_sh.at[pl.ds(sid * RPT, RPT)],
                    out_hbm.at[cid, pl.ds(sid * RPT, RPT)])


def _edge_call():
    return pl.kernel(
        _edge_body, mesh=_mesh(),
    out_type=jax.ShapeDtypeStruct((NC, NACC, HID), jnp.float32),
    compiler_params=pltpu.CompilerParams(use_tc_tiling_on_sc=False),
    scratch_types=[
        pltpu.VMEM((CPW0, CHUNK), jnp.int32),
        pltpu.VMEM((CPW0, CHUNK), jnp.int32),
        pltpu.VMEM((CPW0, CHUNK), jnp.int32),
        pltpu.VMEM((CHUNK, HID), jnp.float32),
        pltpu.VMEM((CHUNK, 2 * HID), jnp.float32),
        pltpu.VMEM((CHUNK, HID), jnp.float32),
        pltpu.VMEM((ZR, HID), jnp.float32),
        pltpu.VMEM_SHARED((NACC, HID), jnp.float32),
        pltpu.SemaphoreType.DMA,
        pltpu.SemaphoreType.DMA,
        pltpu.SemaphoreType.DMA,
        pltpu.SemaphoreType.DMA,
    ],
)


def _tables_common(h, cnt_ref, wall_ref, bf_ref, lin_ref, f_ref, skip_ref):
    t = jnp.dot(h, wall_ref[...], preferred_element_type=jnp.float32)
    lin_ref[...] = t[:, :NR * HID]
    fs = t[:, 768:896]
    sk = t[:, 896:960]
    skip_ref[...] = jnp.maximum(fs[:, HID:] * sk + fs[:, :HID], 0.0)
    cnt = cnt_ref[0] + cnt_ref[1]
    for r in range(NR):
        invr = 1.0 / jnp.maximum(cnt[:, r:r + 1], 1.0)
        lo = NR * HID + r * 2 * HID
        f_ref[:, r * 2 * HID:(r + 1) * 2 * HID] = (
            t[:, lo:lo + 2 * HID] + bf_ref[0, r * 2 * HID:(r + 1) * 2 * HID]
        ) * invr


def _layer0_body(x_ref, cnt_ref, wenc_ref, benc_ref, wall_ref, bf_ref,
                 lin_ref, f_ref, skip_ref):
    h = jnp.dot(x_ref[...], wenc_ref[...],
                preferred_element_type=jnp.float32) + benc_ref[...]
    _tables_common(h, cnt_ref, wall_ref, bf_ref, lin_ref, f_ref, skip_ref)


def _layer1_body(skin_ref, acc_ref, cnt_ref, bns_ref, bnb_ref, wall_ref,
                 bf_ref, lin_ref, f_ref, skip_ref):
    h = skin_ref[...] + acc_ref[0] + acc_ref[1]
    h = jnp.maximum(h * bns_ref[...] + bnb_ref[...], 0.0)
    _tables_common(h, cnt_ref, wall_ref, bf_ref, lin_ref, f_ref, skip_ref)


def _head_body(skin_ref, acc_ref, batch_ref, wl_ref, bl_ref, wc_ref, bc_ref,
               out_ref, accm_ref):
    i = pl.program_id(0)

    @pl.when(i == 0)
    def _():
        accm_ref[...] = jnp.zeros((NG, HID), jnp.float32)

    y = skin_ref[...] + acc_ref[0] + acc_ref[1]
    b = batch_ref[0]
    oh = (b == lax.broadcasted_iota(jnp.int32, (NG, BN), 0)).astype(jnp.float32)
    accm_ref[...] += jnp.dot(oh, y, preferred_element_type=jnp.float32)

    @pl.when(i == NB - 1)
    def _():
        g = jnp.maximum(
            jnp.dot(accm_ref[...], wl_ref[...],
                    preferred_element_type=jnp.float32) + bl_ref[...], 0.0)
        out_ref[...] = jnp.dot(g, wc_ref[...],
                               preferred_element_type=jnp.float32) + bc_ref[...]


def _fixed(shape):
    return pl.BlockSpec(shape, lambda i: tuple(0 for _ in shape))


_tables_out = [
    jax.ShapeDtypeStruct((NN, NR * HID), jnp.float32),
    jax.ShapeDtypeStruct((NN, NR * 2 * HID), jnp.float32),
    jax.ShapeDtypeStruct((NN, HID), jnp.float32),
]
_tables_out_specs = [
    pl.BlockSpec((BN, NR * HID), lambda i: (i, 0)),
    pl.BlockSpec((BN, NR * 2 * HID), lambda i: (i, 0)),
    pl.BlockSpec((BN, HID), lambda i: (i, 0)),
]

_layer0_call = pl.pallas_call(
    _layer0_body,
    grid=(NB,),
    in_specs=[
        pl.BlockSpec((BN, DIN), lambda i: (i, 0)),
        pl.BlockSpec((NC, BN, 4), lambda i: (0, i, 0)),
        _fixed((DIN, HID)),
        _fixed((1, HID)),
        _fixed((HID, 960)),
        _fixed((1, NR * 2 * HID)),
    ],
    out_specs=_tables_out_specs,
    out_shape=_tables_out,
)

_layer1_call = pl.pallas_call(
    _layer1_body,
    grid=(NB,),
    in_specs=[
        pl.BlockSpec((BN, HID), lambda i: (i, 0)),
        pl.BlockSpec((NC, BN, HID), lambda i: (0, i, 0)),
        pl.BlockSpec((NC, BN, 4), lambda i: (0, i, 0)),
        _fixed((1, HID)),
        _fixed((1, HID)),
        _fixed((HID, 960)),
        _fixed((1, NR * 2 * HID)),
    ],
    out_specs=_tables_out_specs,
    out_shape=_tables_out,
)

_head_call = pl.pallas_call(
    _head_body,
    grid=(NB,),
    in_specs=[
        pl.BlockSpec((BN, HID), lambda i: (i, 0)),
        pl.BlockSpec((NC, BN, HID), lambda i: (0, i, 0)),
        pl.BlockSpec((1, 1, BN), lambda i: (i, 0, 0)),
        _fixed((HID, HID)),
        _fixed((1, HID)),
        _fixed((HID, NOUT)),
        _fixed((1, NOUT)),
    ],
    out_specs=pl.BlockSpec((NG, NOUT), lambda i: (0, 0)),
    out_shape=jax.ShapeDtypeStruct((NG, NOUT), jnp.float32),
    scratch_shapes=[pltpu.VMEM((NG, HID), jnp.float32)],
)


def _wall(p, l):
    wcat = jnp.concatenate([p['W_rel_%d' % l][r] for r in range(NR)], axis=1)
    wfcat = jnp.concatenate([p['Wf_rel_%d' % l][r] for r in range(NR)], axis=1)
    wall = jnp.concatenate(
        [wcat, wfcat, p['Wf_skip_%d' % l], p['W_skip_%d' % l]], axis=1)
    bfcat = jnp.concatenate([p['bf_rel_%d' % l][r] for r in range(NR)], axis=0)
    return wall, bfcat.reshape(1, NR * 2 * HID)


def kernel(x, edge_index, edge_type, batch, params):
    p = params
    src = edge_index[0].astype(jnp.int32)
    dst = edge_index[1].astype(jnp.int32)
    et = edge_type.astype(jnp.int32)

    pad = EPAD - EE
    zpad = jnp.zeros((pad,), jnp.int32)
    xg = jnp.concatenate([src * NR + et, zpad]).reshape(TOTCH, CHUNK)
    fg = jnp.concatenate([dst * NR + et, zpad]).reshape(TOTCH, CHUNK)
    d3 = jnp.concatenate([dst, jnp.full((pad,), NN, jnp.int32)]
                         ).reshape(TOTCH, CHUNK)
    dpad = jnp.concatenate([dst, jnp.full((pad,), NN, jnp.int32)])
    etpad = jnp.concatenate([et, zpad])
    fgc3 = (dpad * NR + etpad).reshape(NW, CPW, CHUNK)

    cntf = _cnt_call()(fgc3)
    cnt2 = cntf[:, :NN * NR, 0].reshape(NC, NN, NR)

    wall0, bf0 = _wall(p, 0)
    wall1, bf1 = _wall(p, 1)
    bns = (p['bn_w'] / jnp.sqrt(p['bn_rv'] + 1e-5)).reshape(1, HID)
    bnb = (p['bn_b'] - p['bn_rm'] * bns[0]).reshape(1, HID)

    lin_t, f_t, skip0 = _layer0_call(
        x, cnt2, p['W_enc'], p['b_enc'].reshape(1, HID), wall0, bf0)
    acc0 = _edge_call()(lin_t.reshape(NN * NR, HID),
                        f_t.reshape(NN * NR, 2 * HID), xg, fg, d3)

    lin_t1, f_t1, skip1 = _layer1_call(
        skip0, acc0[:, :NN, :HID], cnt2, bns, bnb, wall1, bf1)
    acc1 = _edge_call()(lin_t1.reshape(NN * NR, HID),
                        f_t1.reshape(NN * NR, 2 * HID), xg, fg, d3)

    out = _head_call(skip1, acc1[:, :NN, :HID],
                     batch.astype(jnp.int32).reshape(NB, 1, BN),
                     p['W_lin'], p['b_lin'].reshape(1, HID),
                     p['W_clf'], p['b_clf'].reshape(1, NOUT))
    return out
